# Initial kernel scaffold; baseline (speedup 1.0000x reference)
import jax
import jax.numpy as jnp
from jax.experimental import pallas as pl


def kernel(edge_index, emb, Wl1, Wr1, att1, b1, Wl2, Wr2, att2, b2):
    raise NotImplementedError("write your pallas kernel here")


# speedup vs baseline: 2.3336x; 2.3336x over previous
import functools

import jax
import jax.numpy as jnp
from jax.experimental import pallas as pl

N = 10000
D = 128


def _mm2_body(x_ref, wl_ref, wr_ref, gl_ref, gr_ref):
    x = x_ref[...]
    gl_ref[...] = jnp.dot(x, wl_ref[...], preferred_element_type=jnp.float32)
    gr_ref[...] = jnp.dot(x, wr_ref[...], preferred_element_type=jnp.float32)


def _mm2(x, Wl, Wr):
    return pl.pallas_call(
        _mm2_body,
        out_shape=[
            jax.ShapeDtypeStruct((N, D), jnp.float32),
            jax.ShapeDtypeStruct((N, D), jnp.float32),
        ],
    )(x, Wl, Wr)


def _gat_layer(x, src, dst, Wl, Wr, att, b):
    gl, gr = _mm2(x, Wl, Wr)
    e = jax.nn.leaky_relu(gl[src] + gr[dst], negative_slope=0.2) @ att
    s = jnp.exp(e)
    denom = jax.ops.segment_sum(s, dst, num_segments=N)
    u = jax.ops.segment_sum(gl[src] * s[:, None], dst, num_segments=N)
    return u / (denom[:, None] + 1e-16) + b


@jax.jit
def kernel(edge_index, emb, Wl1, Wr1, att1, b1, Wl2, Wr2, att2, b2):
    src = edge_index[0]
    dst = edge_index[1]
    h = _gat_layer(emb, src, dst, Wl1, Wr1, att1, b1)
    h = jax.nn.relu(h + emb)
    out = _gat_layer(h, src, dst, Wl2, Wr2, att2, b2)
    return out + h


# speedup vs baseline: 12.4800x; 5.3480x over previous
import functools

import jax
import jax.numpy as jnp
from jax import lax
from jax.experimental import pallas as pl
from jax.experimental.pallas import tpu as pltpu
from jax.experimental.pallas import tpu_sc as plsc

N = 10000
D = 128
E = 320000
C = 128
NCHUNK = E // C
NPAD = 10240
RPT = NPAD // 16
NW = 32
JMAX = (NCHUNK + NW - 1) // NW

_mesh = plsc.VectorSubcoreMesh(core_axis_name="c", subcore_axis_name="s")
_gdn = lax.GatherDimensionNumbers(
    offset_dims=(), collapsed_slice_dims=(0,), start_index_map=(0,))


@functools.partial(
    pl.kernel,
    mesh=_mesh,
    out_type=[
        jax.ShapeDtypeStruct((2, NPAD, D), jnp.float32),
        jax.ShapeDtypeStruct((2, NPAD), jnp.float32),
    ],
    scratch_types=[
        pltpu.VMEM((C,), jnp.int32),
        pltpu.VMEM((C,), jnp.int32),
        pltpu.VMEM((C, D), jnp.float32),
        pltpu.VMEM((C, D), jnp.float32),
        pltpu.VMEM((C + 16,), jnp.float32),
        pltpu.VMEM((D,), jnp.float32),
        pltpu.VMEM_SHARED((NPAD, D), jnp.float32),
        pltpu.VMEM_SHARED((NPAD,), jnp.float32),
        pltpu.SemaphoreType.DMA,
        pltpu.SemaphoreType.DMA,
    ],
)
def _sc_edge_phase(gl_hbm, gr_hbm, ei_hbm, att_hbm, u_hbm, d_hbm,
                   si, di, ga, gb, sb, av, u_sh, d_sh, sem1, sem2):
    cid = lax.axis_index("c")
    sid = lax.axis_index("s")
    wid = sid * 2 + cid

    def run():
        zv = jnp.zeros((16,), jnp.float32)

        def zrow(r, carry):
            for k in range(8):
                ga[r, pl.ds(16 * k, 16)] = zv
            return carry
        lax.fori_loop(0, C, zrow, 0)
        for k in range((C + 16) // 16):
            sb[pl.ds(16 * k, 16)] = zv

        base = sid * RPT
        pltpu.sync_copy(ga, u_sh.at[pl.ds(base, C)])
        for r in range(1, RPT // C):
            pltpu.sync_copy(ga, u_sh.at[pl.ds(base + C * r, C)])
        for r in range(RPT // C):
            pltpu.sync_copy(sb.at[pl.ds(0, C)], d_sh.at[pl.ds(base + C * r, C)])
        plsc.subcore_barrier()

        pltpu.sync_copy(att_hbm, av)
        att_v = [av[pl.ds(16 * k, 16)] for k in range(8)]
        lane_eq = [lax.iota(jnp.int32, 16) == j for j in range(16)]
        perms = [(lax.iota(jnp.int32, 16) ^ sh)[:, None] for sh in (1, 2, 4, 8)]

        def chunk(j, carry):
            t = j * NW + wid

            @pl.when(t < NCHUNK)
            def _():
                e0 = t * C
                pltpu.sync_copy(ei_hbm.at[0, pl.ds(e0, C)], si)
                pltpu.sync_copy(ei_hbm.at[1, pl.ds(e0, C)], di)
                cp1 = pltpu.async_copy(gl_hbm.at[si], ga, sem1)
                cp2 = pltpu.async_copy(gr_hbm.at[di], gb, sem2)
                cp1.wait()
                cp2.wait()

                def group(g, carry2):
                    i0 = g * 16
                    s_carry = zv
                    for j in range(16):
                        i = i0 + j
                        a = [ga[i, pl.ds(16 * k, 16)] for k in range(8)]
                        acc = zv
                        for k in range(8):
                            u = a[k] + gb[i, pl.ds(16 * k, 16)]
                            acc = acc + jnp.maximum(u, 0.2 * u) * att_v[k]
                        for p in perms:
                            acc = acc + lax.gather(
                                acc, p, _gdn, (1,),
                                mode=lax.GatherScatterMode.PROMISE_IN_BOUNDS)
                        sv = jnp.exp(acc)
                        for k in range(8):
                            ga[i, pl.ds(16 * k, 16)] = a[k] * sv
                        s_carry = jnp.where(lane_eq[j], sv, s_carry)
                    sb[pl.ds(i0, 16)] = s_carry
                    return carry2
                lax.fori_loop(0, C // 16, group, 0)

                pltpu.sync_copy(ga, u_sh.at[di], add=True)
                pltpu.sync_copy(sb.at[pl.ds(0, C)], d_sh.at[di], add=True)
            return carry
        lax.fori_loop(0, JMAX, chunk, 0)

        plsc.subcore_barrier()
        pltpu.sync_copy(u_sh.at[pl.ds(base, RPT)], u_hbm.at[cid, pl.ds(base, RPT)])
        pltpu.sync_copy(d_sh.at[pl.ds(base, RPT)], d_hbm.at[cid, pl.ds(base, RPT)])

    run()


def _mm2_body(x_ref, wl_ref, wr_ref, gl_ref, gr_ref):
    x = x_ref[...]
    gl_ref[...] = jnp.dot(x, wl_ref[...], preferred_element_type=jnp.float32)
    gr_ref[...] = jnp.dot(x, wr_ref[...], preferred_element_type=jnp.float32)


def _mm2(x, Wl, Wr):
    return pl.pallas_call(
        _mm2_body,
        out_shape=[
            jax.ShapeDtypeStruct((N, D), jnp.float32),
            jax.ShapeDtypeStruct((N, D), jnp.float32),
        ],
    )(x, Wl, Wr)


def _mid_body(u_ref, d_ref, b_ref, emb_ref, wl_ref, wr_ref,
              h_ref, gl_ref, gr_ref):
    u = u_ref[0] + u_ref[1]
    den = d_ref[0] + d_ref[1]
    h = u / (den[:, None] + 1e-16) + b_ref[...] + emb_ref[...]
    h = jnp.maximum(h, 0.0)
    h_ref[...] = h
    gl_ref[...] = jnp.dot(h, wl_ref[...], preferred_element_type=jnp.float32)
    gr_ref[...] = jnp.dot(h, wr_ref[...], preferred_element_type=jnp.float32)


def _fin_body(u_ref, d_ref, b_ref, h_ref, out_ref):
    u = u_ref[0] + u_ref[1]
    den = d_ref[0] + d_ref[1]
    out_ref[...] = u / (den[:, None] + 1e-16) + b_ref[...] + h_ref[...]


@jax.jit
def kernel(edge_index, emb, Wl1, Wr1, att1, b1, Wl2, Wr2, att2, b2):
    gl1, gr1 = _mm2(emb, Wl1, Wr1)
    u1, d1 = _sc_edge_phase(gl1, gr1, edge_index, att1)
    u1 = u1[:, :N, :]
    d1 = d1[:, :N]
    h, gl2, gr2 = pl.pallas_call(
        _mid_body,
        out_shape=[
            jax.ShapeDtypeStruct((N, D), jnp.float32),
            jax.ShapeDtypeStruct((N, D), jnp.float32),
            jax.ShapeDtypeStruct((N, D), jnp.float32),
        ],
    )(u1, d1, b1[None, :], emb, Wl2, Wr2)
    u2, d2 = _sc_edge_phase(gl2, gr2, edge_index, att2)
    u2 = u2[:, :N, :]
    d2 = d2[:, :N]
    out = pl.pallas_call(
        _fin_body,
        out_shape=jax.ShapeDtypeStruct((N, D), jnp.float32),
    )(u2, d2, b2[None, :], h)
    return out


# speedup vs baseline: 13.2731x; 1.0636x over previous
import functools

import jax
import jax.numpy as jnp
from jax import lax
from jax.experimental import pallas as pl
from jax.experimental.pallas import tpu as pltpu
from jax.experimental.pallas import tpu_sc as plsc

N = 10000
D = 128
E = 320000
C = 64
NCHUNK = E // C
NPAD = 10240
RPT = NPAD // 16
NW = 32
JMAX = (NCHUNK + NW - 1) // NW

_mesh = plsc.VectorSubcoreMesh(core_axis_name="c", subcore_axis_name="s")
_gdn = lax.GatherDimensionNumbers(
    offset_dims=(), collapsed_slice_dims=(0,), start_index_map=(0,))


@functools.partial(
    pl.kernel,
    mesh=_mesh,
    out_type=[
        jax.ShapeDtypeStruct((2, NPAD, D), jnp.float32),
        jax.ShapeDtypeStruct((2, NPAD), jnp.float32),
    ],
    scratch_types=[
        pltpu.VMEM((2, C), jnp.int32),
        pltpu.VMEM((2, C), jnp.int32),
        pltpu.VMEM((2, C), jnp.int32),
        pltpu.VMEM((2, C, D), jnp.float32),
        pltpu.VMEM((2, C, D), jnp.float32),
        pltpu.VMEM((2, C), jnp.float32),
        pltpu.VMEM((D,), jnp.float32),
        pltpu.VMEM_SHARED((NPAD, D), jnp.float32),
        pltpu.VMEM_SHARED((NPAD,), jnp.float32),
        pltpu.SemaphoreType.DMA,
        pltpu.SemaphoreType.DMA,
        pltpu.SemaphoreType.DMA,
    ],
)
def _sc_edge_phase(gl_hbm, gr_hbm, ei_hbm, att_hbm, u_hbm, d_hbm,
                   si, di, dsc, ga, gb, sb, av, u_sh, d_sh,
                   isem, gsem, ssem):
    cid = lax.axis_index("c")
    sid = lax.axis_index("s")
    wid = sid * 2 + cid
    tw = NCHUNK // NW + jnp.where(wid < NCHUNK - (NCHUNK // NW) * NW, 1, 0)

    zv = jnp.zeros((16,), jnp.float32)
    lane_eq = [lax.iota(jnp.int32, 16) == j for j in range(16)]
    perms = [(lax.iota(jnp.int32, 16) ^ sh)[:, None] for sh in (1, 2, 4, 8)]

    def zrow(r, carry):
        for k in range(8):
            ga[0, r, pl.ds(16 * k, 16)] = zv
        return carry
    lax.fori_loop(0, C, zrow, 0)
    for k in range(C // 16):
        sb[0, pl.ds(16 * k, 16)] = zv
    base = sid * RPT
    for r in range(RPT // C):
        pltpu.sync_copy(ga.at[0], u_sh.at[pl.ds(base + C * r, C)])
        pltpu.sync_copy(sb.at[0], d_sh.at[pl.ds(base + C * r, C)])
    plsc.subcore_barrier()

    pltpu.sync_copy(att_hbm, av)
    att_v = [av[pl.ds(16 * k, 16)] for k in range(8)]

    def idx_copy(j, p):
        e0 = (j * NW + wid) * C
        pltpu.async_copy(ei_hbm.at[0, pl.ds(e0, C)], si.at[p], isem)
        pltpu.async_copy(ei_hbm.at[1, pl.ds(e0, C)], di.at[p], isem)

    def wait_idx(p):
        pltpu.make_async_copy(ei_hbm.at[0, pl.ds(0, C)], si.at[p], isem).wait()
        pltpu.make_async_copy(ei_hbm.at[1, pl.ds(0, C)], di.at[p], isem).wait()

    def gather(p):
        pltpu.async_copy(gl_hbm.at[si.at[p]], ga.at[p], gsem)
        pltpu.async_copy(gr_hbm.at[di.at[p]], gb.at[p], gsem)

    def wait_gather(p):
        pltpu.make_async_copy(gl_hbm.at[si.at[p]], ga.at[p], gsem).wait()
        pltpu.make_async_copy(gr_hbm.at[di.at[p]], gb.at[p], gsem).wait()

    def scatter(p):
        pltpu.async_copy(ga.at[p], u_sh.at[dsc.at[p]], ssem, add=True)
        pltpu.async_copy(sb.at[p], d_sh.at[dsc.at[p]], ssem, add=True)

    def wait_scatter(p):
        pltpu.make_async_copy(ga.at[p], u_sh.at[dsc.at[p]], ssem).wait()
        pltpu.make_async_copy(sb.at[p], d_sh.at[dsc.at[p]], ssem).wait()

    e0 = wid * C
    pltpu.sync_copy(ei_hbm.at[0, pl.ds(e0, C)], si.at[0])
    pltpu.sync_copy(ei_hbm.at[1, pl.ds(e0, C)], di.at[0])
    gather(0)
    idx_copy(1, 1)

    def compute_chunk(p):
        for k in range(C // 16):
            dsc[p, pl.ds(16 * k, 16)] = di[p, pl.ds(16 * k, 16)]

        def group(g, carry2):
            i0 = g * 16
            s_carry = zv
            for j2 in range(16):
                i = i0 + j2
                a = [ga[p, i, pl.ds(16 * k, 16)] for k in range(8)]
                acc = zv
                for k in range(8):
                    u = a[k] + gb[p, i, pl.ds(16 * k, 16)]
                    acc = acc + jnp.maximum(u, 0.2 * u) * att_v[k]
                for prm in perms:
                    acc = acc + lax.gather(
                        acc, prm, _gdn, (1,),
                        mode=lax.GatherScatterMode.PROMISE_IN_BOUNDS)
                sv = jnp.exp(acc)
                for k in range(8):
                    ga[p, i, pl.ds(16 * k, 16)] = a[k] * sv
                s_carry = jnp.where(lane_eq[j2], sv, s_carry)
            sb[p, pl.ds(i0, 16)] = s_carry
            return carry2
        lax.fori_loop(0, C // 16, group, 0)

    def step_body(j, p):
        wait_gather(p)
        compute_chunk(p)

        @pl.when(j >= 1)
        def _():
            wait_scatter(1 - p)
        scatter(p)

        @pl.when(j + 1 < tw)
        def _():
            wait_idx(1 - p)
            gather(1 - p)
            @pl.when(j + 2 < tw)
            def _():
                idx_copy(j + 2, p)

    def pair(jj, carry):
        step_body(2 * jj, 0)
        step_body(2 * jj + 1, 1)
        return carry
    lax.fori_loop(0, JMAX // 2, pair, 0)

    @pl.when(tw > 2 * (JMAX // 2))
    def _():
        step_body(2 * (JMAX // 2), 0)
        wait_scatter(0)

    @pl.when(tw <= 2 * (JMAX // 2))
    def _():
        wait_scatter(1)
    plsc.subcore_barrier()
    pltpu.sync_copy(u_sh.at[pl.ds(base, RPT)], u_hbm.at[cid, pl.ds(base, RPT)])
    pltpu.sync_copy(d_sh.at[pl.ds(base, RPT)], d_hbm.at[cid, pl.ds(base, RPT)])


def _mm2_body(x_ref, wl_ref, wr_ref, gl_ref, gr_ref):
    x = x_ref[...]
    gl_ref[...] = jnp.dot(x, wl_ref[...], preferred_element_type=jnp.float32)
    gr_ref[...] = jnp.dot(x, wr_ref[...], preferred_element_type=jnp.float32)


def _mm2(x, Wl, Wr):
    return pl.pallas_call(
        _mm2_body,
        out_shape=[
            jax.ShapeDtypeStruct((N, D), jnp.float32),
            jax.ShapeDtypeStruct((N, D), jnp.float32),
        ],
    )(x, Wl, Wr)


def _mid_body(u_ref, d_ref, b_ref, emb_ref, wl_ref, wr_ref,
              h_ref, gl_ref, gr_ref):
    u = u_ref[0] + u_ref[1]
    den = d_ref[0] + d_ref[1]
    h = u / (den[:, None] + 1e-16) + b_ref[...] + emb_ref[...]
    h = jnp.maximum(h, 0.0)
    h_ref[...] = h
    gl_ref[...] = jnp.dot(h, wl_ref[...], preferred_element_type=jnp.float32)
    gr_ref[...] = jnp.dot(h, wr_ref[...], preferred_element_type=jnp.float32)


def _fin_body(u_ref, d_ref, b_ref, h_ref, out_ref):
    u = u_ref[0] + u_ref[1]
    den = d_ref[0] + d_ref[1]
    out_ref[...] = u / (den[:, None] + 1e-16) + b_ref[...] + h_ref[...]


@jax.jit
def kernel(edge_index, emb, Wl1, Wr1, att1, b1, Wl2, Wr2, att2, b2):
    gl1, gr1 = _mm2(emb, Wl1, Wr1)
    u1, d1 = _sc_edge_phase(gl1, gr1, edge_index, att1)
    u1 = u1[:, :N, :]
    d1 = d1[:, :N]
    h, gl2, gr2 = pl.pallas_call(
        _mid_body,
        out_shape=[
            jax.ShapeDtypeStruct((N, D), jnp.float32),
            jax.ShapeDtypeStruct((N, D), jnp.float32),
            jax.ShapeDtypeStruct((N, D), jnp.float32),
        ],
    )(u1, d1, b1[None, :], emb, Wl2, Wr2)
    u2, d2 = _sc_edge_phase(gl2, gr2, edge_index, att2)
    u2 = u2[:, :N, :]
    d2 = d2[:, :N]
    out = pl.pallas_call(
        _fin_body,
        out_shape=jax.ShapeDtypeStruct((N, D), jnp.float32),
    )(u2, d2, b2[None, :], h)
    return out


# speedup vs baseline: 18.9212x; 1.4255x over previous
import functools

import jax
import jax.numpy as jnp
from jax import lax
from jax.experimental import pallas as pl
from jax.experimental.pallas import tpu as pltpu
from jax.experimental.pallas import tpu_sc as plsc

N = 10000
D = 128
E = 320000
C = 64
NCHUNK = E // C
NPAD = 10240
RPT = NPAD // 16
NW = 32
JMAX = (NCHUNK + NW - 1) // NW

_mesh = plsc.VectorSubcoreMesh(core_axis_name="c", subcore_axis_name="s")
_gdn = lax.GatherDimensionNumbers(
    offset_dims=(), collapsed_slice_dims=(0,), start_index_map=(0,))


@functools.partial(
    pl.kernel,
    mesh=_mesh,
    out_type=[
        jax.ShapeDtypeStruct((2, NPAD, D), jnp.float32),
        jax.ShapeDtypeStruct((2, NPAD), jnp.float32),
    ],
    scratch_types=[
        pltpu.VMEM((2, C), jnp.int32),
        pltpu.VMEM((2, C), jnp.int32),
        pltpu.VMEM((2, C), jnp.int32),
        pltpu.VMEM((2, C, D), jnp.float32),
        pltpu.VMEM((2, C, D), jnp.float32),
        pltpu.VMEM((2, C), jnp.float32),
        pltpu.VMEM((D,), jnp.float32),
        pltpu.VMEM_SHARED((NPAD, D), jnp.float32),
        pltpu.VMEM_SHARED((NPAD,), jnp.float32),
        pltpu.SemaphoreType.DMA,
        pltpu.SemaphoreType.DMA,
        pltpu.SemaphoreType.DMA,
    ],
)
def _sc_edge_phase(gl_hbm, gr_hbm, ei_hbm, att_hbm, u_hbm, d_hbm,
                   si, di, dsc, ga, gb, sb, av, u_sh, d_sh,
                   isem, gsem, ssem):
    cid = lax.axis_index("c")
    sid = lax.axis_index("s")
    wid = sid * 2 + cid
    tw = NCHUNK // NW + jnp.where(wid < NCHUNK - (NCHUNK // NW) * NW, 1, 0)

    zv = jnp.zeros((16,), jnp.float32)
    lane_eq = [lax.iota(jnp.int32, 16) == j for j in range(16)]
    perms = [(lax.iota(jnp.int32, 16) ^ sh)[:, None] for sh in (1, 2, 4, 8)]

    def zrow(r, carry):
        for k in range(8):
            ga[0, r, pl.ds(16 * k, 16)] = zv
        return carry
    lax.fori_loop(0, C, zrow, 0)
    for k in range(C // 16):
        sb[0, pl.ds(16 * k, 16)] = zv
    base = sid * RPT
    for r in range(RPT // C):
        pltpu.sync_copy(ga.at[0], u_sh.at[pl.ds(base + C * r, C)])
        pltpu.sync_copy(sb.at[0], d_sh.at[pl.ds(base + C * r, C)])
    plsc.subcore_barrier()

    pltpu.sync_copy(att_hbm, av)
    att_v = [av[pl.ds(16 * k, 16)] for k in range(8)]

    def idx_copy(j, p):
        e0 = (j * NW + wid) * C
        pltpu.async_copy(ei_hbm.at[0, pl.ds(e0, C)], si.at[p], isem)
        pltpu.async_copy(ei_hbm.at[1, pl.ds(e0, C)], di.at[p], isem)

    def wait_idx(p):
        pltpu.make_async_copy(ei_hbm.at[0, pl.ds(0, C)], si.at[p], isem).wait()
        pltpu.make_async_copy(ei_hbm.at[1, pl.ds(0, C)], di.at[p], isem).wait()

    def gather(p):
        pltpu.async_copy(gl_hbm.at[si.at[p]], ga.at[p], gsem)
        pltpu.async_copy(gr_hbm.at[di.at[p]], gb.at[p], gsem)

    def wait_gather(p):
        pltpu.make_async_copy(gl_hbm.at[si.at[p]], ga.at[p], gsem).wait()
        pltpu.make_async_copy(gr_hbm.at[di.at[p]], gb.at[p], gsem).wait()

    def scatter(p):
        pltpu.async_copy(ga.at[p], u_sh.at[dsc.at[p]], ssem, add=True)
        pltpu.async_copy(sb.at[p], d_sh.at[dsc.at[p]], ssem, add=True)

    def wait_scatter(p):
        pltpu.make_async_copy(ga.at[p], u_sh.at[dsc.at[p]], ssem).wait()
        pltpu.make_async_copy(sb.at[p], d_sh.at[dsc.at[p]], ssem).wait()

    e0 = wid * C
    pltpu.sync_copy(ei_hbm.at[0, pl.ds(e0, C)], si.at[0])
    pltpu.sync_copy(ei_hbm.at[1, pl.ds(e0, C)], di.at[0])
    gather(0)
    idx_copy(1, 1)

    def compute_chunk(p):
        for k in range(C // 16):
            dsc[p, pl.ds(16 * k, 16)] = di[p, pl.ds(16 * k, 16)]

        def group(g):
            i0 = g * 16
            s_carry = zv
            for j2 in range(16):
                i = i0 + j2
                a = [ga[p, i, pl.ds(16 * k, 16)] for k in range(8)]
                acc = zv
                for k in range(8):
                    u = a[k] + gb[p, i, pl.ds(16 * k, 16)]
                    acc = acc + jnp.maximum(u, 0.2 * u) * att_v[k]
                for prm in perms:
                    acc = acc + lax.gather(
                        acc, prm, _gdn, (1,),
                        mode=lax.GatherScatterMode.PROMISE_IN_BOUNDS)
                sv = jnp.exp(acc)
                for k in range(8):
                    ga[p, i, pl.ds(16 * k, 16)] = a[k] * sv
                s_carry = jnp.where(lane_eq[j2], sv, s_carry)
            sb[p, pl.ds(i0, 16)] = s_carry
        plsc.parallel_loop(0, C // 16)(group)

    def step_body(j, p):
        wait_gather(p)

        @pl.when(j >= 1)
        def _():
            wait_scatter(1 - p)

        @pl.when(j + 1 < tw)
        def _():
            wait_idx(1 - p)
            gather(1 - p)

        compute_chunk(p)
        scatter(p)

        @pl.when(j + 2 < tw)
        def _():
            idx_copy(j + 2, p)

    def pair(jj, carry):
        step_body(2 * jj, 0)
        step_body(2 * jj + 1, 1)
        return carry
    lax.fori_loop(0, JMAX // 2, pair, 0)

    @pl.when(tw > 2 * (JMAX // 2))
    def _():
        step_body(2 * (JMAX // 2), 0)
        wait_scatter(0)

    @pl.when(tw <= 2 * (JMAX // 2))
    def _():
        wait_scatter(1)
    plsc.subcore_barrier()
    pltpu.sync_copy(u_sh.at[pl.ds(base, RPT)], u_hbm.at[cid, pl.ds(base, RPT)])
    pltpu.sync_copy(d_sh.at[pl.ds(base, RPT)], d_hbm.at[cid, pl.ds(base, RPT)])


def _mm2_body(x_ref, wl_ref, wr_ref, gl_ref, gr_ref):
    x = x_ref[...]
    gl_ref[...] = jnp.dot(x, wl_ref[...], preferred_element_type=jnp.float32)
    gr_ref[...] = jnp.dot(x, wr_ref[...], preferred_element_type=jnp.float32)


def _mm2(x, Wl, Wr):
    return pl.pallas_call(
        _mm2_body,
        out_shape=[
            jax.ShapeDtypeStruct((N, D), jnp.float32),
            jax.ShapeDtypeStruct((N, D), jnp.float32),
        ],
    )(x, Wl, Wr)


def _mid_body(u_ref, d_ref, b_ref, emb_ref, wl_ref, wr_ref,
              h_ref, gl_ref, gr_ref):
    u = u_ref[0] + u_ref[1]
    den = d_ref[0] + d_ref[1]
    h = u / (den[:, None] + 1e-16) + b_ref[...] + emb_ref[...]
    h = jnp.maximum(h, 0.0)
    h_ref[...] = h
    gl_ref[...] = jnp.dot(h, wl_ref[...], preferred_element_type=jnp.float32)
    gr_ref[...] = jnp.dot(h, wr_ref[...], preferred_element_type=jnp.float32)


def _fin_body(u_ref, d_ref, b_ref, h_ref, out_ref):
    u = u_ref[0] + u_ref[1]
    den = d_ref[0] + d_ref[1]
    out_ref[...] = u / (den[:, None] + 1e-16) + b_ref[...] + h_ref[...]


@jax.jit
def kernel(edge_index, emb, Wl1, Wr1, att1, b1, Wl2, Wr2, att2, b2):
    gl1, gr1 = _mm2(emb, Wl1, Wr1)
    u1, d1 = _sc_edge_phase(gl1, gr1, edge_index, att1)
    u1 = u1[:, :N, :]
    d1 = d1[:, :N]
    h, gl2, gr2 = pl.pallas_call(
        _mid_body,
        out_shape=[
            jax.ShapeDtypeStruct((N, D), jnp.float32),
            jax.ShapeDtypeStruct((N, D), jnp.float32),
            jax.ShapeDtypeStruct((N, D), jnp.float32),
        ],
    )(u1, d1, b1[None, :], emb, Wl2, Wr2)
    u2, d2 = _sc_edge_phase(gl2, gr2, edge_index, att2)
    u2 = u2[:, :N, :]
    d2 = d2[:, :N]
    out = pl.pallas_call(
        _fin_body,
        out_shape=jax.ShapeDtypeStruct((N, D), jnp.float32),
    )(u2, d2, b2[None, :], h)
    return out


# speedup vs baseline: 18.9900x; 1.0036x over previous
import functools

import jax
import jax.numpy as jnp
from jax import lax
from jax.experimental import pallas as pl
from jax.experimental.pallas import tpu as pltpu
from jax.experimental.pallas import tpu_sc as plsc

N = 10000
D = 128
E = 320000
C = 64
NCHUNK = E // C
NPAD = 10240
RPT = NPAD // 16
NW = 32
JMAX = (NCHUNK + NW - 1) // NW

_mesh = plsc.VectorSubcoreMesh(core_axis_name="c", subcore_axis_name="s")
_gdn = lax.GatherDimensionNumbers(
    offset_dims=(), collapsed_slice_dims=(0,), start_index_map=(0,))


@functools.partial(
    pl.kernel,
    mesh=_mesh,
    out_type=[
        jax.ShapeDtypeStruct((2, NPAD, D), jnp.float32),
        jax.ShapeDtypeStruct((2, NPAD), jnp.float32),
    ],
    scratch_types=[
        pltpu.VMEM((2, 2 * C), jnp.int32),
        pltpu.VMEM((2, C), jnp.int32),
        pltpu.VMEM((2, 2 * C, D), jnp.float32),
        pltpu.VMEM((2, C), jnp.float32),
        pltpu.VMEM((D,), jnp.float32),
        pltpu.VMEM_SHARED((NPAD, D), jnp.float32),
        pltpu.VMEM_SHARED((NPAD,), jnp.float32),
        pltpu.SemaphoreType.DMA,
        pltpu.SemaphoreType.DMA,
        pltpu.SemaphoreType.DMA,
    ],
)
def _sc_edge_phase(glr_hbm, ei2_hbm, att_hbm, u_hbm, d_hbm,
                   sdi, dsc, gab, sb, av, u_sh, d_sh,
                   isem, gsem, ssem):
    cid = lax.axis_index("c")
    sid = lax.axis_index("s")
    wid = sid * 2 + cid
    tw = NCHUNK // NW + jnp.where(wid < NCHUNK - (NCHUNK // NW) * NW, 1, 0)

    zv = jnp.zeros((16,), jnp.float32)
    lane_eq = [lax.iota(jnp.int32, 16) == j for j in range(16)]
    perms = [(lax.iota(jnp.int32, 16) ^ sh)[:, None] for sh in (1, 2, 4, 8)]

    def zrow(r, carry):
        for k in range(8):
            gab[0, r, pl.ds(16 * k, 16)] = zv
        return carry
    lax.fori_loop(0, C, zrow, 0)
    for k in range(C // 16):
        sb[0, pl.ds(16 * k, 16)] = zv
    base = sid * RPT
    for r in range(RPT // C):
        pltpu.sync_copy(gab.at[0, pl.ds(0, C)], u_sh.at[pl.ds(base + C * r, C)])
        pltpu.sync_copy(sb.at[0], d_sh.at[pl.ds(base + C * r, C)])
    plsc.subcore_barrier()

    pltpu.sync_copy(att_hbm, av)
    att_v = [av[pl.ds(16 * k, 16)] for k in range(8)]

    def idx_copy(j, p):
        e0 = (j * NW + wid) * 2 * C
        pltpu.async_copy(ei2_hbm.at[pl.ds(e0, 2 * C)], sdi.at[p], isem)

    def wait_idx(p):
        pltpu.make_async_copy(ei2_hbm.at[pl.ds(0, 2 * C)], sdi.at[p], isem).wait()

    def gather(p):
        pltpu.async_copy(glr_hbm.at[sdi.at[p]], gab.at[p], gsem)

    def wait_gather(p):
        pltpu.make_async_copy(glr_hbm.at[sdi.at[p]], gab.at[p], gsem).wait()

    def scatter(p):
        pltpu.async_copy(gab.at[p, pl.ds(0, C)], u_sh.at[dsc.at[p]], ssem,
                         add=True)
        pltpu.async_copy(sb.at[p], d_sh.at[dsc.at[p]], ssem, add=True)

    def wait_scatter(p):
        pltpu.make_async_copy(gab.at[p, pl.ds(0, C)], u_sh.at[dsc.at[p]],
                              ssem).wait()
        pltpu.make_async_copy(sb.at[p], d_sh.at[dsc.at[p]], ssem).wait()

    e0 = wid * 2 * C
    pltpu.sync_copy(ei2_hbm.at[pl.ds(e0, 2 * C)], sdi.at[0])
    gather(0)
    idx_copy(1, 1)

    def compute_chunk(p):
        for k in range(C // 16):
            dsc[p, pl.ds(16 * k, 16)] = sdi[p, pl.ds(C + 16 * k, 16)] - N

        def group(g):
            i0 = g * 16
            s_carry = zv
            for j2 in range(16):
                i = i0 + j2
                a = [gab[p, i, pl.ds(16 * k, 16)] for k in range(8)]
                acc = zv
                for k in range(8):
                    u = a[k] + gab[p, C + i, pl.ds(16 * k, 16)]
                    acc = acc + jnp.maximum(u, 0.2 * u) * att_v[k]
                for prm in perms:
                    acc = acc + lax.gather(
                        acc, prm, _gdn, (1,),
                        mode=lax.GatherScatterMode.PROMISE_IN_BOUNDS)
                sv = jnp.exp(acc)
                for k in range(8):
                    gab[p, i, pl.ds(16 * k, 16)] = a[k] * sv
                s_carry = jnp.where(lane_eq[j2], sv, s_carry)
            sb[p, pl.ds(i0, 16)] = s_carry
        plsc.parallel_loop(0, C // 16)(group)

    def step_body(j, p):
        wait_gather(p)

        @pl.when(j >= 1)
        def _():
            wait_scatter(1 - p)

        @pl.when(j + 1 < tw)
        def _():
            wait_idx(1 - p)
            gather(1 - p)

        compute_chunk(p)
        scatter(p)

        @pl.when(j + 2 < tw)
        def _():
            idx_copy(j + 2, p)

    def pair(jj, carry):
        step_body(2 * jj, 0)
        step_body(2 * jj + 1, 1)
        return carry
    lax.fori_loop(0, JMAX // 2, pair, 0)

    @pl.when(tw > 2 * (JMAX // 2))
    def _():
        step_body(2 * (JMAX // 2), 0)
        wait_scatter(0)

    @pl.when(tw <= 2 * (JMAX // 2))
    def _():
        wait_scatter(1)
    plsc.subcore_barrier()
    pltpu.sync_copy(u_sh.at[pl.ds(base, RPT)], u_hbm.at[cid, pl.ds(base, RPT)])
    pltpu.sync_copy(d_sh.at[pl.ds(base, RPT)], d_hbm.at[cid, pl.ds(base, RPT)])


def _mm2_body(x_ref, wl_ref, wr_ref, glr_ref):
    x = x_ref[...]
    glr_ref[0:N, :] = jnp.dot(x, wl_ref[...], preferred_element_type=jnp.float32)
    glr_ref[N:2 * N, :] = jnp.dot(x, wr_ref[...], preferred_element_type=jnp.float32)


def _mm2(x, Wl, Wr):
    return pl.pallas_call(
        _mm2_body,
        out_shape=jax.ShapeDtypeStruct((2 * N, D), jnp.float32),
    )(x, Wl, Wr)


def _mid_body(u_ref, d_ref, b_ref, emb_ref, wl_ref, wr_ref,
              h_ref, glr_ref):
    u = u_ref[0] + u_ref[1]
    den = d_ref[0] + d_ref[1]
    h = u / (den[:, None] + 1e-16) + b_ref[...] + emb_ref[...]
    h = jnp.maximum(h, 0.0)
    h_ref[...] = h
    glr_ref[0:N, :] = jnp.dot(h, wl_ref[...], preferred_element_type=jnp.float32)
    glr_ref[N:2 * N, :] = jnp.dot(h, wr_ref[...], preferred_element_type=jnp.float32)


def _fin_body(u_ref, d_ref, b_ref, h_ref, out_ref):
    u = u_ref[0] + u_ref[1]
    den = d_ref[0] + d_ref[1]
    out_ref[...] = u / (den[:, None] + 1e-16) + b_ref[...] + h_ref[...]


@jax.jit
def kernel(edge_index, emb, Wl1, Wr1, att1, b1, Wl2, Wr2, att2, b2):
    ei2 = edge_index + jnp.array([[0], [N]], jnp.int32)
    ei2 = jnp.transpose(ei2.reshape(2, NCHUNK, C), (1, 0, 2)).reshape(-1)
    glr1 = _mm2(emb, Wl1, Wr1)
    u1, d1 = _sc_edge_phase(glr1, ei2, att1)
    u1 = u1[:, :N, :]
    d1 = d1[:, :N]
    h, glr2 = pl.pallas_call(
        _mid_body,
        out_shape=[
            jax.ShapeDtypeStruct((N, D), jnp.float32),
            jax.ShapeDtypeStruct((2 * N, D), jnp.float32),
        ],
    )(u1, d1, b1[None, :], emb, Wl2, Wr2)
    u2, d2 = _sc_edge_phase(glr2, ei2, att2)
    u2 = u2[:, :N, :]
    d2 = d2[:, :N]
    out = pl.pallas_call(
        _fin_body,
        out_shape=jax.ShapeDtypeStruct((N, D), jnp.float32),
    )(u2, d2, b2[None, :], h)
    return out


# speedup vs baseline: 19.0230x; 1.0017x over previous
import functools

import jax
import jax.numpy as jnp
from jax import lax
from jax.experimental import pallas as pl
from jax.experimental.pallas import tpu as pltpu
from jax.experimental.pallas import tpu_sc as plsc

N = 10000
D = 128
E = 320000
C = 64
NCHUNK = E // C
NPAD = 10240
RPT = NPAD // 16
NW = 32
JMAX = (NCHUNK + NW - 1) // NW

_mesh = plsc.VectorSubcoreMesh(core_axis_name="c", subcore_axis_name="s")
_gdn = lax.GatherDimensionNumbers(
    offset_dims=(), collapsed_slice_dims=(0,), start_index_map=(0,))


@functools.partial(
    pl.kernel,
    mesh=_mesh,
    out_type=[
        jax.ShapeDtypeStruct((2, NPAD, D), jnp.float32),
        jax.ShapeDtypeStruct((2, NPAD), jnp.float32),
    ],
    scratch_types=[
        pltpu.VMEM((2, 2 * C), jnp.int32),
        pltpu.VMEM((2, C), jnp.int32),
        pltpu.VMEM((2, 2 * C, D), jnp.float32),
        pltpu.VMEM((2, C), jnp.float32),
        pltpu.VMEM((D,), jnp.float32),
        pltpu.VMEM_SHARED((NPAD, D), jnp.float32),
        pltpu.VMEM_SHARED((NPAD,), jnp.float32),
        pltpu.SemaphoreType.DMA,
        pltpu.SemaphoreType.DMA,
        pltpu.SemaphoreType.DMA,
    ],
)
def _sc_edge_phase(glr_hbm, ei2_hbm, att_hbm, u_hbm, d_hbm,
                   sdi, dsc, gab, sb, av, u_sh, d_sh,
                   isem, gsem, ssem):
    cid = lax.axis_index("c")
    sid = lax.axis_index("s")
    wid = sid * 2 + cid
    tw = NCHUNK // NW + jnp.where(wid < NCHUNK - (NCHUNK // NW) * NW, 1, 0)

    zv = jnp.zeros((16,), jnp.float32)
    lane_eq = [lax.iota(jnp.int32, 16) == j for j in range(16)]
    perms = [(lax.iota(jnp.int32, 16) ^ sh)[:, None] for sh in (1, 2, 4, 8)]


    def idx_copy(j, p):
        e0 = (j * NW + wid) * 2 * C
        pltpu.async_copy(ei2_hbm.at[pl.ds(e0, 2 * C)], sdi.at[p], isem)

    def wait_idx(p):
        pltpu.make_async_copy(ei2_hbm.at[pl.ds(0, 2 * C)], sdi.at[p], isem).wait()

    def gather(p):
        pltpu.async_copy(glr_hbm.at[sdi.at[p]], gab.at[p], gsem)

    def wait_gather(p):
        pltpu.make_async_copy(glr_hbm.at[sdi.at[p]], gab.at[p], gsem).wait()

    def scatter(p):
        pltpu.async_copy(gab.at[p, pl.ds(0, C)], u_sh.at[dsc.at[p]], ssem,
                         add=True)
        pltpu.async_copy(sb.at[p], d_sh.at[dsc.at[p]], ssem, add=True)

    def wait_scatter(p):
        pltpu.make_async_copy(gab.at[p, pl.ds(0, C)], u_sh.at[dsc.at[p]],
                              ssem).wait()
        pltpu.make_async_copy(sb.at[p], d_sh.at[dsc.at[p]], ssem).wait()

    e0 = wid * 2 * C
    pltpu.sync_copy(ei2_hbm.at[pl.ds(e0, 2 * C)], sdi.at[0])
    gather(0)
    idx_copy(1, 1)

    pltpu.sync_copy(att_hbm, av)
    att_v = [av[pl.ds(16 * k, 16)] for k in range(8)]

    def zrow(r, carry):
        for k in range(8):
            gab[1, r, pl.ds(16 * k, 16)] = zv
        return carry
    lax.fori_loop(0, C, zrow, 0)
    for k in range(C // 16):
        sb[1, pl.ds(16 * k, 16)] = zv
    base = sid * RPT
    for r in range(RPT // C):
        pltpu.sync_copy(gab.at[1, pl.ds(0, C)], u_sh.at[pl.ds(base + C * r, C)])
        pltpu.sync_copy(sb.at[1], d_sh.at[pl.ds(base + C * r, C)])
    plsc.subcore_barrier()

    def compute_chunk(p):
        for k in range(C // 16):
            dsc[p, pl.ds(16 * k, 16)] = sdi[p, pl.ds(C + 16 * k, 16)] - N

        def group(g):
            i0 = g * 16
            s_carry = zv
            for j2 in range(16):
                i = i0 + j2
                a = [gab[p, i, pl.ds(16 * k, 16)] for k in range(8)]
                acc = zv
                for k in range(8):
                    u = a[k] + gab[p, C + i, pl.ds(16 * k, 16)]
                    acc = acc + jnp.maximum(u, 0.2 * u) * att_v[k]
                for prm in perms:
                    acc = acc + lax.gather(
                        acc, prm, _gdn, (1,),
                        mode=lax.GatherScatterMode.PROMISE_IN_BOUNDS)
                sv = jnp.exp(acc)
                for k in range(8):
                    gab[p, i, pl.ds(16 * k, 16)] = a[k] * sv
                s_carry = jnp.where(lane_eq[j2], sv, s_carry)
            sb[p, pl.ds(i0, 16)] = s_carry
        plsc.parallel_loop(0, C // 16)(group)

    def step_body(j, p):
        wait_gather(p)

        @pl.when(j >= 1)
        def _():
            wait_scatter(1 - p)

        @pl.when(j + 1 < tw)
        def _():
            wait_idx(1 - p)
            gather(1 - p)

        compute_chunk(p)
        scatter(p)

        @pl.when(j + 2 < tw)
        def _():
            idx_copy(j + 2, p)

    def pair(jj, carry):
        step_body(2 * jj, 0)
        step_body(2 * jj + 1, 1)
        return carry
    lax.fori_loop(0, JMAX // 2, pair, 0)

    @pl.when(tw > 2 * (JMAX // 2))
    def _():
        step_body(2 * (JMAX // 2), 0)
        wait_scatter(0)

    @pl.when(tw <= 2 * (JMAX // 2))
    def _():
        wait_scatter(1)
    plsc.subcore_barrier()
    pltpu.sync_copy(u_sh.at[pl.ds(base, RPT)], u_hbm.at[cid, pl.ds(base, RPT)])
    pltpu.sync_copy(d_sh.at[pl.ds(base, RPT)], d_hbm.at[cid, pl.ds(base, RPT)])


def _mm2_body(x_ref, wl_ref, wr_ref, glr_ref):
    x = x_ref[...]
    glr_ref[0:N, :] = jnp.dot(x, wl_ref[...], preferred_element_type=jnp.float32)
    glr_ref[N:2 * N, :] = jnp.dot(x, wr_ref[...], preferred_element_type=jnp.float32)


def _mm2(x, Wl, Wr):
    return pl.pallas_call(
        _mm2_body,
        out_shape=jax.ShapeDtypeStruct((2 * N, D), jnp.float32),
    )(x, Wl, Wr)


def _mid_body(u_ref, d_ref, b_ref, emb_ref, wl_ref, wr_ref,
              h_ref, glr_ref):
    u = u_ref[0] + u_ref[1]
    den = d_ref[0] + d_ref[1]
    h = u / (den[:, None] + 1e-16) + b_ref[...] + emb_ref[...]
    h = jnp.maximum(h, 0.0)
    h_ref[...] = h
    glr_ref[0:N, :] = jnp.dot(h, wl_ref[...], preferred_element_type=jnp.float32)
    glr_ref[N:2 * N, :] = jnp.dot(h, wr_ref[...], preferred_element_type=jnp.float32)


def _fin_body(u_ref, d_ref, b_ref, h_ref, out_ref):
    u = u_ref[0] + u_ref[1]
    den = d_ref[0] + d_ref[1]
    out_ref[...] = u / (den[:, None] + 1e-16) + b_ref[...] + h_ref[...]


@jax.jit
def kernel(edge_index, emb, Wl1, Wr1, att1, b1, Wl2, Wr2, att2, b2):
    ei2 = edge_index + jnp.array([[0], [N]], jnp.int32)
    ei2 = jnp.transpose(ei2.reshape(2, NCHUNK, C), (1, 0, 2)).reshape(-1)
    glr1 = _mm2(emb, Wl1, Wr1)
    u1, d1 = _sc_edge_phase(glr1, ei2, att1)
    u1 = u1[:, :N, :]
    d1 = d1[:, :N]
    h, glr2 = pl.pallas_call(
        _mid_body,
        out_shape=[
            jax.ShapeDtypeStruct((N, D), jnp.float32),
            jax.ShapeDtypeStruct((2 * N, D), jnp.float32),
        ],
    )(u1, d1, b1[None, :], emb, Wl2, Wr2)
    u2, d2 = _sc_edge_phase(glr2, ei2, att2)
    u2 = u2[:, :N, :]
    d2 = d2[:, :N]
    out = pl.pallas_call(
        _fin_body,
        out_shape=jax.ShapeDtypeStruct((N, D), jnp.float32),
    )(u2, d2, b2[None, :], h)
    return out
